# baseline (speedup 1.0000x reference)
import functools

import jax
import jax.numpy as jnp
from jax.experimental import pallas as pl
from jax.experimental.pallas import tpu as pltpu

_HOP = 1920
_D = 512
_K = 2048
_Q = 8
_TILE = 512


def _rvq_body(frames_ref, w_ref, b_ref, cbh_ref, cbm_ref, cbl_ref,
              codes_ref, res_ref, cn_ref):
    i = pl.program_id(0)
    q = pl.program_id(1)

    @pl.when(q == 0)
    def _init():
        lat = jnp.dot(frames_ref[...].astype(jnp.bfloat16),
                      w_ref[...].astype(jnp.bfloat16),
                      preferred_element_type=jnp.float32)
        res_ref[...] = lat + b_ref[...]

    cbh = cbh_ref[0]

    @pl.when(i == 0)
    def _cnorm():
        cb = ((cbh.astype(jnp.float32) + cbm_ref[0].astype(jnp.float32))
              + cbl_ref[0].astype(jnp.float32))
        cn_ref[pl.ds(q, 1), :] = jnp.sum(cb * cb, axis=1)[None, :]

    r = res_ref[...]
    rnorm = jnp.sum(r * r, axis=1, keepdims=True)
    scores = jax.lax.dot_general(
        r.astype(jnp.bfloat16), cbh, (((1,), (1,)), ((), ())),
        preferred_element_type=jnp.float32)
    d = rnorm - 2.0 * scores + cn_ref[pl.ds(q, 1), :]

    iota = jax.lax.broadcasted_iota(jnp.int32, (_TILE, _K), 1)
    minv = jnp.min(d, axis=1, keepdims=True)
    idx = jnp.min(jnp.where(d == minv, iota, _K), axis=1)
    codes_ref[0, 0, :] = idx

    onehot = (iota == idx[:, None]).astype(jnp.bfloat16)
    g = lambda part_ref: jax.lax.dot_general(
        onehot, part_ref[0], (((1,), (0,)), ((), ())),
        preferred_element_type=jnp.float32)
    quant = (g(cbh_ref) + g(cbm_ref)) + g(cbl_ref)
    res_ref[...] = r - quant


@functools.partial(jax.jit, static_argnames=())
def kernel(audio_input, W_enc, b_enc, codebooks):
    B = audio_input.shape[0]
    x = audio_input.reshape(B, -1)
    T = x.shape[1] // _HOP
    rows = B * T
    frames = x[:, : T * _HOP].reshape(rows, _HOP)
    n_tiles = (rows + _TILE - 1) // _TILE
    padded = n_tiles * _TILE
    if padded != rows:
        frames = jnp.concatenate(
            [frames, jnp.zeros((padded - rows, _HOP), jnp.float32)], axis=0)

    cb_hi = jax.lax.optimization_barrier(codebooks.astype(jnp.bfloat16))
    rem = codebooks - cb_hi.astype(jnp.float32)
    cb_mid = jax.lax.optimization_barrier(rem.astype(jnp.bfloat16))
    cb_lo = (rem - cb_mid.astype(jnp.float32)).astype(jnp.bfloat16)

    codes = pl.pallas_call(
        _rvq_body,
        grid=(n_tiles, _Q),
        in_specs=[
            pl.BlockSpec((_TILE, _HOP), lambda i, q: (i, 0)),
            pl.BlockSpec((_HOP, _D), lambda i, q: (0, 0)),
            pl.BlockSpec((1, _D), lambda i, q: (0, 0)),
            pl.BlockSpec((1, _K, _D), lambda i, q: (q, 0, 0)),
            pl.BlockSpec((1, _K, _D), lambda i, q: (q, 0, 0)),
            pl.BlockSpec((1, _K, _D), lambda i, q: (q, 0, 0)),
        ],
        out_specs=pl.BlockSpec(
            (1, 1, _TILE), lambda i, q, nt=n_tiles: (q * nt + i, 0, 0)),
        out_shape=jax.ShapeDtypeStruct((_Q * n_tiles, 1, _TILE), jnp.int32),
        scratch_shapes=[pltpu.VMEM((_TILE, _D), jnp.float32),
                        pltpu.VMEM((_Q, _K), jnp.float32)],
    )(frames, W_enc, b_enc.reshape(1, _D), cb_hi, cb_mid, cb_lo)

    codes = codes.reshape(_Q, padded)[:, :rows]
    codes = codes.reshape(_Q, B, T).transpose(1, 0, 2)
    return codes.astype(jnp.int32)


# speedup vs baseline: 1.0737x; 1.0737x over previous
import functools

import jax
import jax.numpy as jnp
from jax.experimental import pallas as pl
from jax.experimental.pallas import tpu as pltpu

_HOP = 1920
_D = 512
_K = 2048
_Q = 8
_TILE = 512


def _rvq_body(frames_ref, w_ref, b_ref, cb_ref, codes_ref, res_ref, cn_ref):
    i = pl.program_id(0)
    q = pl.program_id(1)

    @pl.when(q == 0)
    def _init():
        lat = jnp.dot(frames_ref[...].astype(jnp.bfloat16),
                      w_ref[...].astype(jnp.bfloat16),
                      preferred_element_type=jnp.float32)
        res_ref[...] = lat + b_ref[...]

    cb = cb_ref[0]

    @pl.when(i == 0)
    def _cnorm():
        cn_ref[pl.ds(q, 1), :] = jnp.sum(cb * cb, axis=1)[None, :]

    r = res_ref[...]
    rnorm = jnp.sum(r * r, axis=1, keepdims=True)
    cb_hi = cb.astype(jnp.bfloat16)
    scores = jax.lax.dot_general(
        r.astype(jnp.bfloat16), cb_hi, (((1,), (1,)), ((), ())),
        preferred_element_type=jnp.float32)
    d = rnorm - 2.0 * scores + cn_ref[pl.ds(q, 1), :]

    iota = jax.lax.broadcasted_iota(jnp.int32, (_TILE, _K), 1)
    minv = jnp.min(d, axis=1, keepdims=True)
    idx = jnp.min(jnp.where(d == minv, iota, _K), axis=1)
    codes_ref[0, 0, :] = idx

    onehot = (iota == idx[:, None]).astype(jnp.bfloat16)
    rem = cb - cb_hi.astype(jnp.float32)
    cb_mid = rem.astype(jnp.bfloat16)
    cb_lo = (rem - cb_mid.astype(jnp.float32)).astype(jnp.bfloat16)
    g = lambda part: jax.lax.dot_general(
        onehot, part, (((1,), (0,)), ((), ())),
        preferred_element_type=jnp.float32)
    quant = (g(cb_hi) + g(cb_mid)) + g(cb_lo)
    res_ref[...] = r - quant


@functools.partial(jax.jit, static_argnames=())
def kernel(audio_input, W_enc, b_enc, codebooks):
    B = audio_input.shape[0]
    x = audio_input.reshape(B, -1)
    T = x.shape[1] // _HOP
    rows = B * T
    frames = x[:, : T * _HOP].reshape(rows, _HOP)
    n_tiles = (rows + _TILE - 1) // _TILE
    padded = n_tiles * _TILE
    if padded != rows:
        frames = jnp.concatenate(
            [frames, jnp.zeros((padded - rows, _HOP), jnp.float32)], axis=0)

    codes = pl.pallas_call(
        _rvq_body,
        grid=(n_tiles, _Q),
        in_specs=[
            pl.BlockSpec((_TILE, _HOP), lambda i, q: (i, 0)),
            pl.BlockSpec((_HOP, _D), lambda i, q: (0, 0)),
            pl.BlockSpec((1, _D), lambda i, q: (0, 0)),
            pl.BlockSpec((1, _K, _D), lambda i, q: (q, 0, 0)),
        ],
        out_specs=pl.BlockSpec(
            (1, 1, _TILE), lambda i, q, nt=n_tiles: (q * nt + i, 0, 0)),
        out_shape=jax.ShapeDtypeStruct((_Q * n_tiles, 1, _TILE), jnp.int32),
        scratch_shapes=[pltpu.VMEM((_TILE, _D), jnp.float32),
                        pltpu.VMEM((_Q, _K), jnp.float32)],
    )(frames, W_enc, b_enc.reshape(1, _D), codebooks)

    codes = codes.reshape(_Q, padded)[:, :rows]
    codes = codes.reshape(_Q, B, T).transpose(1, 0, 2)
    return codes.astype(jnp.int32)


# speedup vs baseline: 1.1037x; 1.0280x over previous
import functools

import jax
import jax.numpy as jnp
from jax.experimental import pallas as pl
from jax.experimental.pallas import tpu as pltpu

_HOP = 1920
_D = 512
_K = 2048
_Q = 8
_TILE = 512


def _rvq_body(frames_ref, w_ref, b_ref, cb_ref, codes_ref,
              res_ref, cbh_ref, cbm_ref, cbl_ref, cn_ref):
    q = pl.program_id(0)
    i = pl.program_id(1)
    rows = pl.ds(i * _TILE, _TILE)

    @pl.when(q == 0)
    def _encode():
        lat = jnp.dot(frames_ref[...].astype(jnp.bfloat16),
                      w_ref[...].astype(jnp.bfloat16),
                      preferred_element_type=jnp.float32)
        res_ref[rows, :] = lat + b_ref[...]

    @pl.when(i == 0)
    def _prep_codebook():
        cb = cb_ref[0]
        cb_hi = cb.astype(jnp.bfloat16)
        rem = cb - cb_hi.astype(jnp.float32)
        cb_mid = rem.astype(jnp.bfloat16)
        cbh_ref[...] = cb_hi
        cbm_ref[...] = cb_mid
        cbl_ref[...] = (rem - cb_mid.astype(jnp.float32)).astype(jnp.bfloat16)
        cn_ref[...] = jnp.sum(cb * cb, axis=1)[None, :]

    r = res_ref[rows, :]
    rnorm = jnp.sum(r * r, axis=1, keepdims=True)
    scores = jax.lax.dot_general(
        r.astype(jnp.bfloat16), cbh_ref[...], (((1,), (1,)), ((), ())),
        preferred_element_type=jnp.float32)
    d = rnorm - 2.0 * scores + cn_ref[...]

    iota = jax.lax.broadcasted_iota(jnp.int32, (_TILE, _K), 1)
    minv = jnp.min(d, axis=1, keepdims=True)
    idx = jnp.min(jnp.where(d == minv, iota, _K), axis=1)
    codes_ref[0, 0, :] = idx

    onehot = (iota == idx[:, None]).astype(jnp.bfloat16)
    g = lambda part_ref: jax.lax.dot_general(
        onehot, part_ref[...], (((1,), (0,)), ((), ())),
        preferred_element_type=jnp.float32)
    quant = (g(cbh_ref) + g(cbm_ref)) + g(cbl_ref)
    res_ref[rows, :] = r - quant


@functools.partial(jax.jit, static_argnames=())
def kernel(audio_input, W_enc, b_enc, codebooks):
    B = audio_input.shape[0]
    x = audio_input.reshape(B, -1)
    T = x.shape[1] // _HOP
    rows = B * T
    frames = x[:, : T * _HOP].reshape(rows, _HOP)
    n_tiles = (rows + _TILE - 1) // _TILE
    padded = n_tiles * _TILE
    if padded != rows:
        frames = jnp.concatenate(
            [frames, jnp.zeros((padded - rows, _HOP), jnp.float32)], axis=0)

    codes = pl.pallas_call(
        _rvq_body,
        grid=(_Q, n_tiles),
        in_specs=[
            pl.BlockSpec((_TILE, _HOP), lambda q, i: (i, 0)),
            pl.BlockSpec((_HOP, _D), lambda q, i: (0, 0)),
            pl.BlockSpec((1, _D), lambda q, i: (0, 0)),
            pl.BlockSpec((1, _K, _D), lambda q, i: (q, 0, 0)),
        ],
        out_specs=pl.BlockSpec(
            (1, 1, _TILE), lambda q, i, nt=n_tiles: (q * nt + i, 0, 0)),
        out_shape=jax.ShapeDtypeStruct((_Q * n_tiles, 1, _TILE), jnp.int32),
        scratch_shapes=[pltpu.VMEM((padded, _D), jnp.float32),
                        pltpu.VMEM((_K, _D), jnp.bfloat16),
                        pltpu.VMEM((_K, _D), jnp.bfloat16),
                        pltpu.VMEM((_K, _D), jnp.bfloat16),
                        pltpu.VMEM((1, _K), jnp.float32)],
    )(frames, W_enc, b_enc.reshape(1, _D), codebooks)

    codes = codes.reshape(_Q, padded)[:, :rows]
    codes = codes.reshape(_Q, B, T).transpose(1, 0, 2)
    return codes.astype(jnp.int32)
